# baseline (device time: 20686 ns/iter reference)
import jax
import jax.numpy as jnp
from jax import lax
from jax.experimental import pallas as pl
from jax.experimental.pallas import tpu as pltpu

N_DEV = 16


def kernel(table, idx):
    v_per, d = table.shape
    n = idx.shape[0]
    m = n // N_DEV
    idx2 = idx.reshape(n, 1)

    def body(idx_ref, table_ref, out_ref, part_ref, rs_buf,
             rs_send_sems, rs_recv_sems, ag_send_sems, ag_recv_sems):
        my = lax.axis_index("i")

        barrier_sem = pltpu.get_barrier_semaphore()
        for r in range(1, N_DEV):
            peer = (my + r) % N_DEV
            pl.semaphore_signal(
                barrier_sem, inc=1,
                device_id=(peer,), device_id_type=pl.DeviceIdType.MESH,
            )
        pl.semaphore_wait(barrier_sem, N_DEV - 1)

        local = idx_ref[:, :] - my * v_per
        col = lax.broadcasted_iota(jnp.int32, (n, v_per), 1)
        oh = (col == local).astype(jnp.bfloat16)
        tb = table_ref[:, :].astype(jnp.bfloat16)
        part_ref[:, :] = jnp.dot(oh, tb, preferred_element_type=jnp.float32)

        rs_sends = []
        for r in range(1, N_DEV):
            peer = (my + r) % N_DEV
            k = N_DEV - r
            rdma = pltpu.make_async_remote_copy(
                src_ref=part_ref.at[pl.ds(peer * m, m), :],
                dst_ref=rs_buf.at[k],
                send_sem=rs_send_sems.at[r - 1],
                recv_sem=rs_recv_sems.at[k - 1],
                device_id=(peer,),
                device_id_type=pl.DeviceIdType.MESH,
            )
            rdma.start()
            rs_sends.append(rdma)

        rs_buf[0] = part_ref[pl.ds(my * m, m), :]

        for k in range(1, N_DEV):
            pltpu.make_async_remote_copy(
                src_ref=rs_buf.at[k],
                dst_ref=rs_buf.at[k],
                send_sem=rs_send_sems.at[k - 1],
                recv_sem=rs_recv_sems.at[k - 1],
                device_id=(my,),
                device_id_type=pl.DeviceIdType.MESH,
            ).wait_recv()
        red = jnp.sum(rs_buf[:, :, :], axis=0)
        out_ref[pl.ds(my * m, m), :] = red

        ag_sends = []
        for r in range(1, N_DEV):
            peer = (my + r) % N_DEV
            k = N_DEV - r
            rdma = pltpu.make_async_remote_copy(
                src_ref=out_ref.at[pl.ds(my * m, m), :],
                dst_ref=out_ref.at[pl.ds(my * m, m), :],
                send_sem=ag_send_sems.at[r - 1],
                recv_sem=ag_recv_sems.at[k - 1],
                device_id=(peer,),
                device_id_type=pl.DeviceIdType.MESH,
            )
            rdma.start()
            ag_sends.append(rdma)

        for k in range(1, N_DEV):
            src = (my + k) % N_DEV
            pltpu.make_async_remote_copy(
                src_ref=out_ref.at[pl.ds(src * m, m), :],
                dst_ref=out_ref.at[pl.ds(src * m, m), :],
                send_sem=ag_send_sems.at[k - 1],
                recv_sem=ag_recv_sems.at[k - 1],
                device_id=(my,),
                device_id_type=pl.DeviceIdType.MESH,
            ).wait_recv()

        for rdma in rs_sends:
            rdma.wait_send()
        for rdma in ag_sends:
            rdma.wait_send()

    return pl.pallas_call(
        body,
        out_shape=jax.ShapeDtypeStruct((n, d), jnp.float32),
        in_specs=[
            pl.BlockSpec(memory_space=pltpu.VMEM),
            pl.BlockSpec(memory_space=pltpu.VMEM),
        ],
        out_specs=pl.BlockSpec(memory_space=pltpu.VMEM),
        scratch_shapes=[
            pltpu.VMEM((n, d), jnp.float32),
            pltpu.VMEM((N_DEV, m, d), jnp.float32),
            pltpu.SemaphoreType.DMA((N_DEV - 1,)),
            pltpu.SemaphoreType.DMA((N_DEV - 1,)),
            pltpu.SemaphoreType.DMA((N_DEV - 1,)),
            pltpu.SemaphoreType.DMA((N_DEV - 1,)),
        ],
        compiler_params=pltpu.CompilerParams(collective_id=0),
    )(idx2, table)


# device time: 20182 ns/iter; 1.0250x vs baseline; 1.0250x over previous
import jax
import jax.numpy as jnp
from jax import lax
from jax.experimental import pallas as pl
from jax.experimental.pallas import tpu as pltpu

N_DEV = 16


def kernel(table, idx):
    v_per, d = table.shape
    n = idx.shape[0]
    m = n // N_DEV
    idx2 = idx.reshape(n, 1)

    def body(idx_ref, table_ref, out_ref, part_ref, rs_buf, ag_buf,
             rs_send_sems, rs_recv_sems, ag_send_sems, ag_recv_sems):
        my = lax.axis_index("i")

        barrier_sem = pltpu.get_barrier_semaphore()
        for r in range(1, N_DEV):
            peer = (my + r) % N_DEV
            pl.semaphore_signal(
                barrier_sem, inc=1,
                device_id=(peer,), device_id_type=pl.DeviceIdType.MESH,
            )
        pl.semaphore_wait(barrier_sem, N_DEV - 1)

        local = idx_ref[:, :] - my * v_per
        col = lax.broadcasted_iota(jnp.int32, (n, v_per), 1)
        oh = (col == local).astype(jnp.bfloat16)
        tb = table_ref[:, :].astype(jnp.bfloat16)
        part_ref[:, :] = jnp.dot(
            oh, tb, preferred_element_type=jnp.float32
        ).astype(jnp.bfloat16)

        rs_sends = []
        for r in range(1, N_DEV):
            peer = (my + r) % N_DEV
            k = N_DEV - r
            rdma = pltpu.make_async_remote_copy(
                src_ref=part_ref.at[pl.ds(peer * m, m), :],
                dst_ref=rs_buf.at[k],
                send_sem=rs_send_sems.at[r - 1],
                recv_sem=rs_recv_sems.at[k - 1],
                device_id=(peer,),
                device_id_type=pl.DeviceIdType.MESH,
            )
            rdma.start()
            rs_sends.append(rdma)

        rs_buf[0] = part_ref[pl.ds(my * m, m), :]

        for k in range(1, N_DEV):
            pltpu.make_async_remote_copy(
                src_ref=rs_buf.at[k],
                dst_ref=rs_buf.at[k],
                send_sem=rs_send_sems.at[k - 1],
                recv_sem=rs_recv_sems.at[k - 1],
                device_id=(my,),
                device_id_type=pl.DeviceIdType.MESH,
            ).wait_recv()
        red = jnp.sum(rs_buf[:, :, :].astype(jnp.float32), axis=0)
        ag_buf[pl.ds(my * m, m), :] = red.astype(jnp.bfloat16)

        ag_sends = []
        for r in range(1, N_DEV):
            peer = (my + r) % N_DEV
            k = N_DEV - r
            rdma = pltpu.make_async_remote_copy(
                src_ref=ag_buf.at[pl.ds(my * m, m), :],
                dst_ref=ag_buf.at[pl.ds(my * m, m), :],
                send_sem=ag_send_sems.at[r - 1],
                recv_sem=ag_recv_sems.at[k - 1],
                device_id=(peer,),
                device_id_type=pl.DeviceIdType.MESH,
            )
            rdma.start()
            ag_sends.append(rdma)

        for k in range(1, N_DEV):
            src = (my + k) % N_DEV
            pltpu.make_async_remote_copy(
                src_ref=ag_buf.at[pl.ds(src * m, m), :],
                dst_ref=ag_buf.at[pl.ds(src * m, m), :],
                send_sem=ag_send_sems.at[k - 1],
                recv_sem=ag_recv_sems.at[k - 1],
                device_id=(my,),
                device_id_type=pl.DeviceIdType.MESH,
            ).wait_recv()
        out_ref[:, :] = ag_buf[:, :].astype(jnp.float32)

        for rdma in rs_sends:
            rdma.wait_send()
        for rdma in ag_sends:
            rdma.wait_send()

    return pl.pallas_call(
        body,
        out_shape=jax.ShapeDtypeStruct((n, d), jnp.float32),
        in_specs=[
            pl.BlockSpec(memory_space=pltpu.VMEM),
            pl.BlockSpec(memory_space=pltpu.VMEM),
        ],
        out_specs=pl.BlockSpec(memory_space=pltpu.VMEM),
        scratch_shapes=[
            pltpu.VMEM((n, d), jnp.bfloat16),
            pltpu.VMEM((N_DEV, m, d), jnp.bfloat16),
            pltpu.VMEM((n, d), jnp.bfloat16),
            pltpu.SemaphoreType.DMA((N_DEV - 1,)),
            pltpu.SemaphoreType.DMA((N_DEV - 1,)),
            pltpu.SemaphoreType.DMA((N_DEV - 1,)),
            pltpu.SemaphoreType.DMA((N_DEV - 1,)),
        ],
        compiler_params=pltpu.CompilerParams(collective_id=0),
    )(idx2, table)


# device time: 16439 ns/iter; 1.2583x vs baseline; 1.2277x over previous
import jax
import jax.numpy as jnp
from jax import lax
from jax.experimental import pallas as pl
from jax.experimental.pallas import tpu as pltpu

N_DEV = 16


def kernel(table, idx):
    v_per, d = table.shape
    n = idx.shape[0]
    m = n // N_DEV
    idx2 = idx.reshape(n, 1)

    def body(idx_ref, table_ref, out_ref, part_ref, rs_buf, ag_buf,
             rs_send_sems, rs_recv_sems, ag_send_sems, ag_recv_sems):
        my = lax.axis_index("i")

        barrier_sem = pltpu.get_barrier_semaphore()
        for r in range(1, N_DEV):
            peer = (my + r) % N_DEV
            pl.semaphore_signal(
                barrier_sem, inc=1,
                device_id=(peer,), device_id_type=pl.DeviceIdType.MESH,
            )

        local = idx_ref[:, :] - my * v_per
        col = lax.broadcasted_iota(jnp.int32, (n, v_per), 1)
        oh = (col == local).astype(jnp.bfloat16)
        tb = table_ref[:, :].astype(jnp.bfloat16)
        part_ref[:, :] = jnp.dot(
            oh, tb, preferred_element_type=jnp.float32
        ).astype(jnp.bfloat16)

        pl.semaphore_wait(barrier_sem, N_DEV - 1)

        rs_sends = []
        for r in range(1, N_DEV):
            peer = (my + r) % N_DEV
            k = N_DEV - r
            rdma = pltpu.make_async_remote_copy(
                src_ref=part_ref.at[pl.ds(peer * m, m), :],
                dst_ref=rs_buf.at[k],
                send_sem=rs_send_sems.at[r - 1],
                recv_sem=rs_recv_sems.at[k - 1],
                device_id=(peer,),
                device_id_type=pl.DeviceIdType.MESH,
            )
            rdma.start()
            rs_sends.append(rdma)

        rs_buf[0] = part_ref[pl.ds(my * m, m), :]

        for k in range(1, N_DEV):
            pltpu.make_async_remote_copy(
                src_ref=rs_buf.at[k],
                dst_ref=rs_buf.at[k],
                send_sem=rs_send_sems.at[k - 1],
                recv_sem=rs_recv_sems.at[k - 1],
                device_id=(my,),
                device_id_type=pl.DeviceIdType.MESH,
            ).wait_recv()
        red = jnp.sum(rs_buf[:, :, :].astype(jnp.float32), axis=0)
        ag_buf[pl.ds(my * m, m), :] = red.astype(jnp.bfloat16)

        ag_sends = []
        for r in range(1, N_DEV):
            peer = (my + r) % N_DEV
            k = N_DEV - r
            rdma = pltpu.make_async_remote_copy(
                src_ref=ag_buf.at[pl.ds(my * m, m), :],
                dst_ref=ag_buf.at[pl.ds(my * m, m), :],
                send_sem=ag_send_sems.at[r - 1],
                recv_sem=ag_recv_sems.at[k - 1],
                device_id=(peer,),
                device_id_type=pl.DeviceIdType.MESH,
            )
            rdma.start()
            ag_sends.append(rdma)

        for k in range(1, N_DEV):
            src = (my + k) % N_DEV
            pltpu.make_async_remote_copy(
                src_ref=ag_buf.at[pl.ds(src * m, m), :],
                dst_ref=ag_buf.at[pl.ds(src * m, m), :],
                send_sem=ag_send_sems.at[k - 1],
                recv_sem=ag_recv_sems.at[k - 1],
                device_id=(my,),
                device_id_type=pl.DeviceIdType.MESH,
            ).wait_recv()
        out_ref[:, :] = ag_buf[:, :].astype(jnp.float32)

        for rdma in rs_sends:
            rdma.wait_send()
        for rdma in ag_sends:
            rdma.wait_send()

    return pl.pallas_call(
        body,
        out_shape=jax.ShapeDtypeStruct((n, d), jnp.float32),
        in_specs=[
            pl.BlockSpec(memory_space=pltpu.VMEM),
            pl.BlockSpec(memory_space=pltpu.VMEM),
        ],
        out_specs=pl.BlockSpec(memory_space=pltpu.VMEM),
        scratch_shapes=[
            pltpu.VMEM((n, d), jnp.bfloat16),
            pltpu.VMEM((N_DEV, m, d), jnp.bfloat16),
            pltpu.VMEM((n, d), jnp.bfloat16),
            pltpu.SemaphoreType.DMA((N_DEV - 1,)),
            pltpu.SemaphoreType.DMA((N_DEV - 1,)),
            pltpu.SemaphoreType.DMA((N_DEV - 1,)),
            pltpu.SemaphoreType.DMA((N_DEV - 1,)),
        ],
        compiler_params=pltpu.CompilerParams(collective_id=0),
    )(idx2, table)


# device time: 16281 ns/iter; 1.2706x vs baseline; 1.0097x over previous
import jax
import jax.numpy as jnp
from jax import lax
from jax.experimental import pallas as pl
from jax.experimental.pallas import tpu as pltpu

N_DEV = 16


def kernel(table, idx):
    v_per, d = table.shape
    n = idx.shape[0]
    m = n // N_DEV
    idx2 = idx.reshape(n, 1)

    def body(idx_ref, table_ref, out_ref, part_ref, rs_buf, ag_buf,
             rs_send_sems, rs_recv_sems, ag_send_sems, ag_recv_sems):
        my = lax.axis_index("i")

        barrier_sem = pltpu.get_barrier_semaphore()
        for r in range(1, N_DEV):
            peer = (my + r) % N_DEV
            pl.semaphore_signal(
                barrier_sem, inc=1,
                device_id=(peer,), device_id_type=pl.DeviceIdType.MESH,
            )

        local = idx_ref[:, :] - my * v_per
        col = lax.broadcasted_iota(jnp.int32, (n, v_per), 1)
        oh = (col == local).astype(jnp.bfloat16)
        tb = table_ref[:, :].astype(jnp.bfloat16)
        part_ref[:, :] = jnp.dot(
            oh, tb, preferred_element_type=jnp.float32
        ).astype(jnp.bfloat16)

        pl.semaphore_wait(barrier_sem, N_DEV - 1)

        rs_sends = []
        for r in range(1, N_DEV):
            peer = (my + r) % N_DEV
            k = N_DEV - r
            rdma = pltpu.make_async_remote_copy(
                src_ref=part_ref.at[pl.ds(peer * m, m), :],
                dst_ref=rs_buf.at[k],
                send_sem=rs_send_sems.at[r - 1],
                recv_sem=rs_recv_sems.at[k - 1],
                device_id=(peer,),
                device_id_type=pl.DeviceIdType.MESH,
            )
            rdma.start()
            rs_sends.append(rdma)

        rs_buf[0] = part_ref[pl.ds(my * m, m), :]

        red = rs_buf[0].astype(jnp.float32)
        for k in range(1, N_DEV):
            pltpu.make_async_remote_copy(
                src_ref=rs_buf.at[k],
                dst_ref=rs_buf.at[k],
                send_sem=rs_send_sems.at[k - 1],
                recv_sem=rs_recv_sems.at[k - 1],
                device_id=(my,),
                device_id_type=pl.DeviceIdType.MESH,
            ).wait_recv()
            red = red + rs_buf[k].astype(jnp.float32)
        ag_buf[pl.ds(my * m, m), :] = red.astype(jnp.bfloat16)
        out_ref[pl.ds(my * m, m), :] = red

        ag_sends = []
        for r in range(1, N_DEV):
            peer = (my + r) % N_DEV
            k = N_DEV - r
            rdma = pltpu.make_async_remote_copy(
                src_ref=ag_buf.at[pl.ds(my * m, m), :],
                dst_ref=ag_buf.at[pl.ds(my * m, m), :],
                send_sem=ag_send_sems.at[r - 1],
                recv_sem=ag_recv_sems.at[k - 1],
                device_id=(peer,),
                device_id_type=pl.DeviceIdType.MESH,
            )
            rdma.start()
            ag_sends.append(rdma)

        for k in range(1, N_DEV):
            src = (my + k) % N_DEV
            pltpu.make_async_remote_copy(
                src_ref=ag_buf.at[pl.ds(src * m, m), :],
                dst_ref=ag_buf.at[pl.ds(src * m, m), :],
                send_sem=ag_send_sems.at[k - 1],
                recv_sem=ag_recv_sems.at[k - 1],
                device_id=(my,),
                device_id_type=pl.DeviceIdType.MESH,
            ).wait_recv()
            out_ref[pl.ds(src * m, m), :] = ag_buf[
                pl.ds(src * m, m), :
            ].astype(jnp.float32)

        for rdma in rs_sends:
            rdma.wait_send()
        for rdma in ag_sends:
            rdma.wait_send()

    return pl.pallas_call(
        body,
        out_shape=jax.ShapeDtypeStruct((n, d), jnp.float32),
        in_specs=[
            pl.BlockSpec(memory_space=pltpu.VMEM),
            pl.BlockSpec(memory_space=pltpu.VMEM),
        ],
        out_specs=pl.BlockSpec(memory_space=pltpu.VMEM),
        scratch_shapes=[
            pltpu.VMEM((n, d), jnp.bfloat16),
            pltpu.VMEM((N_DEV, m, d), jnp.bfloat16),
            pltpu.VMEM((n, d), jnp.bfloat16),
            pltpu.SemaphoreType.DMA((N_DEV - 1,)),
            pltpu.SemaphoreType.DMA((N_DEV - 1,)),
            pltpu.SemaphoreType.DMA((N_DEV - 1,)),
            pltpu.SemaphoreType.DMA((N_DEV - 1,)),
        ],
        compiler_params=pltpu.CompilerParams(collective_id=0),
    )(idx2, table)
